# baseline (speedup 1.0000x reference)
import jax
import jax.numpy as jnp
from jax import lax
from jax.experimental import pallas as pl
from jax.experimental.pallas import tpu as pltpu, tpu_sc as plsc

NUM_CORES = 2
NUM_SUBCORES = 16
LANES = 16
NW = NUM_CORES * NUM_SUBCORES

BATCH = 16384
EMBED_DIM = 32
CHUNK = BATCH // NW
SUB = 128
NSUB = CHUNK // SUB
GROUPS = CHUNK // LANES


def _body(uidx_hbm, iidx_hbm, uemb_hbm, iemb_hbm, ubias_hbm, ibias_hbm,
          gb_hbm, out_hbm,
          idx_u, idx_i, rows_u, rows_i, bias_u, bias_i, gb_v, out_v, sem):
    c = lax.axis_index("c")
    s = lax.axis_index("s")
    wid = s * NUM_CORES + c
    base = wid * NSUB

    pltpu.sync_copy(uidx_hbm.at[pl.ds(base, NSUB)], idx_u)
    pltpu.sync_copy(iidx_hbm.at[pl.ds(base, NSUB)], idx_i)
    pltpu.sync_copy(gb_hbm, gb_v)

    copies = []
    for j in range(NSUB):
        copies.append(pltpu.async_copy(
            uemb_hbm.at[idx_u.at[j]], rows_u.at[pl.ds(j * SUB, SUB)], sem))
        copies.append(pltpu.async_copy(
            iemb_hbm.at[idx_i.at[j]], rows_i.at[pl.ds(j * SUB, SUB)], sem))
        copies.append(pltpu.async_copy(
            ubias_hbm.at[idx_u.at[j]], bias_u.at[pl.ds(j * SUB, SUB)], sem))
        copies.append(pltpu.async_copy(
            ibias_hbm.at[idx_i.at[j]], bias_i.at[pl.ds(j * SUB, SUB)], sem))
    for cp in copies:
        cp.wait()

    gb = gb_v[...]
    lanes = lax.iota(jnp.int32, LANES)

    def group(g, carry):
        rb = g * LANES
        acc = gb + bias_u[pl.ds(rb, LANES)] + bias_i[pl.ds(rb, LANES)]
        row_idx = rb + lanes
        for d in range(EMBED_DIM):
            col = jnp.full((LANES,), d, jnp.int32)
            cu = plsc.load_gather(rows_u, [row_idx, col])
            ci = plsc.load_gather(rows_i, [row_idx, col])
            acc = acc + cu * ci
        out_v[pl.ds(rb, LANES)] = acc
        return carry

    lax.fori_loop(0, GROUPS, group, 0)
    pltpu.sync_copy(out_v, out_hbm.at[pl.ds(wid * CHUNK, CHUNK)])


@jax.jit
def kernel(user_idx, item_idx, user_emb, item_emb, user_bias, item_bias,
           global_bias):
    uidx2 = user_idx.astype(jnp.int32).reshape(BATCH // SUB, SUB)
    iidx2 = item_idx.astype(jnp.int32).reshape(BATCH // SUB, SUB)
    ubias = user_bias.reshape(-1)
    ibias = item_bias.reshape(-1)
    gb16 = jnp.broadcast_to(global_bias, (LANES,))

    mesh = plsc.VectorSubcoreMesh(core_axis_name="c", subcore_axis_name="s",
                                  num_cores=NUM_CORES,
                                  num_subcores=NUM_SUBCORES)
    f = pl.kernel(
        _body,
        out_type=jax.ShapeDtypeStruct((BATCH,), jnp.float32),
        mesh=mesh,
        scratch_types=[
            pltpu.VMEM((NSUB, SUB), jnp.int32),
            pltpu.VMEM((NSUB, SUB), jnp.int32),
            pltpu.VMEM((CHUNK, EMBED_DIM), jnp.float32),
            pltpu.VMEM((CHUNK, EMBED_DIM), jnp.float32),
            pltpu.VMEM((CHUNK,), jnp.float32),
            pltpu.VMEM((CHUNK,), jnp.float32),
            pltpu.VMEM((LANES,), jnp.float32),
            pltpu.VMEM((CHUNK,), jnp.float32),
            pltpu.SemaphoreType.DMA,
        ],
        compiler_params=pltpu.CompilerParams(needs_layout_passes=False,
                                             use_tc_tiling_on_sc=False),
    )
    return f(uidx2, iidx2, user_emb, item_emb, ubias, ibias, gb16)


# speedup vs baseline: 3.3330x; 3.3330x over previous
import jax
import jax.numpy as jnp
from jax import lax
from jax.experimental import pallas as pl
from jax.experimental.pallas import tpu as pltpu, tpu_sc as plsc

NUM_CORES = 2
NUM_SUBCORES = 16
LANES = 16
NW = NUM_CORES * NUM_SUBCORES

BATCH = 16384
D = 32
CHUNK = BATCH // NW
NROWS = 1000000

CHUNK_R = 1024
CPW = 30
LO_X16 = CPW * NW * CHUNK_R
LO_EXTRA = LO_X16 + 16 * CHUNK_R
LO_TAIL = LO_EXTRA + 512
TAIL_W = NROWS - LO_TAIL

SLOT_CAP = 768
POS_MAX = SLOT_CAP - LANES
CAND_PAD = SLOT_CAP + LANES
DUMP_BASE = BATCH


def _wid():
    c = lax.axis_index("c")
    s = lax.axis_index("s")
    return s * NUM_CORES + c


def _gather_pass(tab_hbm, tail_hbm, idx_hbm, out1_hbm,
                 idx_v, cand_r, cand_b, moff, mslot, buf_a, buf_b, buf_t,
                 stage, sem_a, sem_b, sem_c, wid):
    lo = wid * CPW * CHUNK_R
    hi = lo + CPW * CHUNK_R
    xlo = LO_X16 + wid * CHUNK_R
    lanes = lax.iota(jnp.int32, LANES)

    pltpu.sync_copy(idx_hbm, idx_v)

    def pre(g, carry):
        cand_b[pl.ds(g * LANES, LANES)] = DUMP_BASE + wid * 8 + (lanes & 7)
        return carry

    lax.fori_loop(0, CAND_PAD // LANES, pre, 0)

    is0 = wid == 0
    is31 = wid == NW - 1
    isx = wid < 16

    def filt(g, pos):
        v = idx_v[pl.ds(g * LANES, LANES)]
        m = (v >= lo) & (v < hi)
        m = m | (isx & (v >= xlo) & (v < xlo + CHUNK_R))
        m = m | (is0 & (v >= LO_EXTRA) & (v < LO_TAIL))
        m = m | (is31 & (v >= LO_TAIL))
        cnt = plsc.all_reduce_population_count(m)[0]

        @pl.when(cnt > 0)
        def _():
            plsc.store_compressed(cand_r.at[pl.ds(pos, LANES)], v, mask=m)
            plsc.store_compressed(cand_b.at[pl.ds(pos, LANES)],
                                  g * LANES + lanes, mask=m)

        return jnp.minimum(pos + cnt, POS_MAX)

    pos = lax.fori_loop(0, BATCH // LANES, filt, jnp.int32(0))
    ngrp = (pos + LANES - 1) // LANES

    def rescan(base, width, buf):
        def cgroup(q, mpos):
            cv = cand_r[pl.ds(q * LANES, LANES)]
            off = cv - base
            m = (off >= 0) & (off < width)
            cnt = plsc.all_reduce_population_count(m)[0]

            @pl.when(cnt > 0)
            def _():
                slotv = jnp.minimum(q * LANES + lanes, SLOT_CAP - 1)
                plsc.store_compressed(moff.at[pl.ds(mpos, LANES)], off, mask=m)
                plsc.store_compressed(mslot.at[pl.ds(mpos, LANES)], slotv,
                                      mask=m)

            return jnp.minimum(mpos + cnt, POS_MAX)

        nm = lax.fori_loop(0, ngrp, cgroup, jnp.int32(0))

        def egroup(e, carry):
            off = moff[pl.ds(e * LANES, LANES)]
            slotb = mslot[pl.ds(e * LANES, LANES)] * D
            m2 = (e * LANES + lanes) < nm
            for d in range(D):
                vals = plsc.load_gather(
                    buf, [jnp.full((LANES,), d, jnp.int32), off], mask=m2)
                plsc.store_scatter(stage, [slotb + d], vals, mask=m2)
            return carry

        lax.fori_loop(0, (nm + LANES - 1) // LANES, egroup, 0)

    def pair(k, carry):
        ba = lo + (2 * k) * CHUNK_R
        bb = lo + (2 * k + 1) * CHUNK_R
        cpa = pltpu.async_copy(tab_hbm.at[:, pl.ds(ba, CHUNK_R)], buf_a, sem_a)
        cpb = pltpu.async_copy(tab_hbm.at[:, pl.ds(bb, CHUNK_R)], buf_b, sem_b)
        cpa.wait()
        rescan(ba, CHUNK_R, buf_a)
        cpb.wait()
        rescan(bb, CHUNK_R, buf_b)
        return carry

    lax.fori_loop(0, CPW // 2, pair, 0)

    @pl.when(isx)
    def _():
        pltpu.sync_copy(tab_hbm.at[:, pl.ds(xlo, CHUNK_R)], buf_a)

    @pl.when(isx)
    def _():
        rescan(xlo, CHUNK_R, buf_a)

    @pl.when(is0)
    def _():
        pltpu.sync_copy(tab_hbm.at[:, pl.ds(LO_EXTRA, 512)],
                        buf_b.at[:, pl.ds(0, 512)])

    @pl.when(is0)
    def _():
        rescan(LO_EXTRA, 512, buf_b)

    @pl.when(is31)
    def _():
        pltpu.sync_copy(tail_hbm, buf_t)

    @pl.when(is31)
    def _():
        rescan(LO_TAIL, TAIL_W, buf_t)

    def sgrp(t, carry):
        bvec = cand_b[pl.ds(t * LANES, LANES)]
        for l in range(LANES):
            b0 = bvec[l]
            pltpu.async_copy(stage.at[pl.ds((t * LANES + l) * D, D)],
                             out1_hbm.at[pl.ds(b0 * D, D)], sem_c)
        return carry

    lax.fori_loop(0, SLOT_CAP // LANES, sgrp, 0)
    pltpu.make_async_copy(out1_hbm.at[pl.ds(0, SLOT_CAP * D)],
                          stage, sem_c).wait()


def _scan_body(uidx_hbm, iidx_hbm, uT_hbm, iT_hbm, uTt_hbm, iTt_hbm,
               U1_hbm, I1_hbm,
               idx_v, cand_r, cand_b, moff, mslot, buf_a, buf_b, buf_t,
               stage, sem_a, sem_b, sem_c):
    wid = _wid()
    _gather_pass(uT_hbm, uTt_hbm, uidx_hbm, U1_hbm,
                 idx_v, cand_r, cand_b, moff, mslot, buf_a, buf_b, buf_t,
                 stage, sem_a, sem_b, sem_c, wid)
    _gather_pass(iT_hbm, iTt_hbm, iidx_hbm, I1_hbm,
                 idx_v, cand_r, cand_b, moff, mslot, buf_a, buf_b, buf_t,
                 stage, sem_a, sem_b, sem_c, wid)


def _dots_body(uidx_hbm, iidx_hbm, U1_hbm, I1_hbm, ubias_hbm, ibias_hbm,
               gb_hbm, out_hbm,
               idx_u, idx_i, idx_u8, idx_i8, u_flat, i_flat,
               bias_u, bias_i, gb_v, out_v, sem):
    wid = _wid()
    base = wid * CHUNK
    lanes = lax.iota(jnp.int32, LANES)

    pltpu.sync_copy(uidx_hbm.at[pl.ds(base, CHUNK)], idx_u)
    pltpu.sync_copy(iidx_hbm.at[pl.ds(base, CHUNK)], idx_i)
    pltpu.sync_copy(U1_hbm.at[pl.ds(base * D, CHUNK * D)], u_flat)
    pltpu.sync_copy(I1_hbm.at[pl.ds(base * D, CHUNK * D)], i_flat)
    pltpu.sync_copy(gb_hbm, gb_v)

    def rows8(g, carry):
        sl = pl.ds(g * LANES, LANES)
        idx_u8[sl] = lax.shift_right_logical(idx_u[sl], 3)
        idx_i8[sl] = lax.shift_right_logical(idx_i[sl], 3)
        return carry

    lax.fori_loop(0, CHUNK // LANES, rows8, 0)

    cps = []
    for j in range(4):
        sl = pl.ds(j * 128, 128)
        cps.append(pltpu.async_copy(
            ubias_hbm.at[idx_u8.at[sl]], bias_u.at[sl], sem))
        cps.append(pltpu.async_copy(
            ibias_hbm.at[idx_i8.at[sl]], bias_i.at[sl], sem))
    for cp in cps:
        cp.wait()

    gb = gb_v[...]

    def group(g, carry):
        sl = pl.ds(g * LANES, LANES)
        slot = g * LANES + lanes
        bu = plsc.load_gather(bias_u, [slot, idx_u[sl] & 7])
        bi = plsc.load_gather(bias_i, [slot, idx_i[sl] & 7])
        acc = gb + bu + bi
        fu = slot * D
        for d in range(D):
            cu = plsc.load_gather(u_flat, [fu + d])
            ci = plsc.load_gather(i_flat, [fu + d])
            acc = acc + cu * ci
        out_v[sl] = acc
        return carry

    lax.fori_loop(0, CHUNK // LANES, group, 0)
    pltpu.sync_copy(out_v, out_hbm.at[pl.ds(wid * CHUNK, CHUNK)])


@jax.jit
def kernel(user_idx, item_idx, user_emb, item_emb, user_bias, item_bias,
           global_bias):
    uidx1 = user_idx.astype(jnp.int32)
    iidx1 = item_idx.astype(jnp.int32)
    uT = user_emb.T
    iT = item_emb.T
    uTt = user_emb[LO_TAIL:].T
    iTt = item_emb[LO_TAIL:].T
    ub8 = user_bias.reshape(NROWS // 8, 8)
    ib8 = item_bias.reshape(NROWS // 8, 8)
    gb16 = jnp.broadcast_to(global_bias, (LANES,))

    mesh = plsc.VectorSubcoreMesh(core_axis_name="c", subcore_axis_name="s",
                                  num_cores=NUM_CORES,
                                  num_subcores=NUM_SUBCORES)

    scan_fn = pl.kernel(
        _scan_body,
        out_type=(jax.ShapeDtypeStruct(((BATCH + 256) * D,), jnp.float32),
                  jax.ShapeDtypeStruct(((BATCH + 256) * D,), jnp.float32)),
        mesh=mesh,
        scratch_types=[
            pltpu.VMEM((BATCH,), jnp.int32),
            pltpu.VMEM((CAND_PAD,), jnp.int32),
            pltpu.VMEM((CAND_PAD,), jnp.int32),
            pltpu.VMEM((CAND_PAD,), jnp.int32),
            pltpu.VMEM((CAND_PAD,), jnp.int32),
            pltpu.VMEM((D, CHUNK_R), jnp.float32),
            pltpu.VMEM((D, CHUNK_R), jnp.float32),
            pltpu.VMEM((D, TAIL_W), jnp.float32),
            pltpu.VMEM((SLOT_CAP * D,), jnp.float32),
            pltpu.SemaphoreType.DMA,
            pltpu.SemaphoreType.DMA,
            pltpu.SemaphoreType.DMA,
        ],
        compiler_params=pltpu.CompilerParams(needs_layout_passes=False,
                                             use_tc_tiling_on_sc=True),
    )
    U1, I1 = scan_fn(uidx1, iidx1, uT, iT, uTt, iTt)

    dots_fn = pl.kernel(
        _dots_body,
        out_type=jax.ShapeDtypeStruct((BATCH,), jnp.float32),
        mesh=mesh,
        scratch_types=[
            pltpu.VMEM((CHUNK,), jnp.int32),
            pltpu.VMEM((CHUNK,), jnp.int32),
            pltpu.VMEM((CHUNK,), jnp.int32),
            pltpu.VMEM((CHUNK,), jnp.int32),
            pltpu.VMEM((CHUNK * D,), jnp.float32),
            pltpu.VMEM((CHUNK * D,), jnp.float32),
            pltpu.VMEM((CHUNK, 8), jnp.float32),
            pltpu.VMEM((CHUNK, 8), jnp.float32),
            pltpu.VMEM((LANES,), jnp.float32),
            pltpu.VMEM((CHUNK,), jnp.float32),
            pltpu.SemaphoreType.DMA,
        ],
        compiler_params=pltpu.CompilerParams(needs_layout_passes=False,
                                             use_tc_tiling_on_sc=False),
    )
    return dots_fn(uidx1, iidx1, U1, I1, ub8, ib8, gb16)
